# baseline (speedup 1.0000x reference)
import functools

import jax
import jax.numpy as jnp
from jax import lax
from jax.experimental import pallas as pl
from jax.experimental.pallas import tpu as pltpu
from jax.experimental.pallas import tpu_sc as plsc

_F32 = jnp.float32

_NC = 2
_NS = 16
_LN = 16
_NW = _NC * _NS
_CHUNK = 128
_NCHUNK = 2


def _mm(a, b):
    return lax.dot_general(a, b, (((1,), (0,)), ((), ())),
                           preferred_element_type=_F32)


def _sc_stage_factory(L, H, off, Bc):
    n_ch = L // _CHUNK
    rows_per_w = Bc // _NW
    mesh = plsc.VectorSubcoreMesh(core_axis_name="c", subcore_axis_name="s")

    @functools.partial(
        pl.kernel,
        out_type=jax.ShapeDtypeStruct((Bc * 5 * L, H), _F32),
        mesh=mesh,
        scratch_types=[
            pltpu.VMEM((n_ch, _CHUNK), jnp.int32),
            pltpu.VMEM((n_ch, _CHUNK), jnp.int32),
            pltpu.VMEM((n_ch, _CHUNK), jnp.int32),
            pltpu.VMEM((n_ch, _CHUNK), jnp.int32),
            pltpu.VMEM((n_ch, _CHUNK), jnp.int32),
            pltpu.VMEM((_CHUNK, H), _F32),
            pltpu.VMEM((_CHUNK, H), _F32),
            pltpu.VMEM((_CHUNK, H), _F32),
            pltpu.VMEM_SHARED((_NS * L, H), _F32),
            pltpu.SemaphoreType.DMA,
            pltpu.SemaphoreType.DMA,
            pltpu.SemaphoreType.DMA,
            pltpu.SemaphoreType.DMA,
            pltpu.SemaphoreType.DMA,
            pltpu.SemaphoreType.DMA,
            pltpu.SemaphoreType.DMA,
            pltpu.SemaphoreType.DMA,
        ],
    )
    def sc_stage(x_hbm, h0_hbm, idxd_hbm, idxr_hbm, idxl_hbm, out_hbm,
                 gidx_d, gidx_r, gidx_l, sidx_r, sidx_l,
                 G0, G1, zbuf, acc,
                 isem, gsemA, gsemB, wsemA, wsemB, zsem, ssemA, ssemB):
        cid = lax.axis_index("c")
        sid = lax.axis_index("s")
        w = sid * _NC + cid
        accbase = sid * L
        G = (G0, G1)
        gsem = (gsemA, gsemB)
        wsem = (wsemA, wsemB)
        ssem = (ssemA, ssemB)

        def zfill(j, carry):
            for k in range(H // _LN):
                zbuf[j, pl.ds(k * _LN, _LN)] = jnp.zeros((_LN,), _F32)
            return carry
        lax.fori_loop(0, _CHUNK, zfill, 0)

        def row_body(t, carry):
            b = off + w * rows_per_w + t
            bL = b * L
            ob = (w * rows_per_w + t) * 5 * L

            i0 = pltpu.async_copy(idxd_hbm.at[b], gidx_d, isem)
            i1 = pltpu.async_copy(idxr_hbm.at[b], gidx_r, isem)
            i2 = pltpu.async_copy(idxl_hbm.at[b], gidx_l, isem)
            i0.wait(); i1.wait(); i2.wait()
            for j in range(n_ch):
                for k in range(_CHUNK // _LN):
                    sl = pl.ds(k * _LN, _LN)
                    sidx_r[j, sl] = gidx_r[j, sl] + accbase
                    sidx_l[j, sl] = gidx_l[j, sl] + accbase
                    gidx_d[j, sl] = gidx_d[j, sl] + bL
                    gidx_r[j, sl] = gidx_r[j, sl] + bL
                    gidx_l[j, sl] = gidx_l[j, sl] + bL

            zs = [pltpu.async_copy(
                      zbuf, acc.at[pl.ds(accbase + ch * _CHUNK, _CHUNK)], zsem)
                  for ch in range(n_ch)]
            for z in zs:
                z.wait()

            def scatter(si):
                sds = {}
                ads = {}
                for ch in range(n_ch):
                    bid = ch % 2
                    if ch >= 2:
                        ads[ch - 2].wait()
                    sds[ch] = pltpu.async_copy(
                        h0_hbm.at[pl.ds(bL + ch * _CHUNK, _CHUNK)],
                        G[bid], gsem[bid])
                    sds[ch].wait()
                    ads[ch] = pltpu.async_copy(
                        G[bid], acc.at[si.at[ch]], ssem[bid], add=True)
                ads[n_ch - 2].wait()
                ads[n_ch - 1].wait()

            scatter(sidx_r)
            ro_r = pltpu.async_copy(acc.at[pl.ds(accbase, L)],
                                    out_hbm.at[pl.ds(ob + 3 * L, L)], wsemA)
            ro_r.wait()
            zs = [pltpu.async_copy(
                      zbuf, acc.at[pl.ds(accbase + ch * _CHUNK, _CHUNK)], zsem)
                  for ch in range(n_ch)]
            for z in zs:
                z.wait()
            scatter(sidx_l)
            ro_l = pltpu.async_copy(acc.at[pl.ds(accbase, L)],
                                    out_hbm.at[pl.ds(ob + 4 * L, L)], wsemB)

            units = []
            for k, (gi, src) in enumerate(((gidx_d, x_hbm), (gidx_r, h0_hbm),
                                           (gidx_l, h0_hbm))):
                for ch in range(n_ch):
                    units.append((gi, src, ob + k * L + ch * _CHUNK))
            gds = {}
            wds = {}

            def writeout(u):
                return pltpu.async_copy(
                    G[u % 2], out_hbm.at[pl.ds(units[u][2], _CHUNK)],
                    wsem[u % 2])

            for u, (gi, src, _) in enumerate(units):
                bid = u % 2
                if u >= 2:
                    wds[u - 2].wait()
                gds[u] = pltpu.async_copy(
                    src.at[gi.at[u % n_ch]], G[bid], gsem[bid])
                if u >= 1:
                    gds[u - 1].wait()
                    wds[u - 1] = writeout(u - 1)
            last = len(units) - 1
            gds[last].wait()
            wds[last] = writeout(last)
            wds[last - 1].wait()
            wds[last].wait()
            ro_l.wait()
            return carry

        lax.fori_loop(0, rows_per_w, row_body, 0)

    return sc_stage


def _cell_body(idx_d_ref, x_ref, h0_ref, c0_ref, g_ref,
               W_ioux_ref, b_ioux_ref, W_iouh0_ref, W_iouh1_ref,
               W_fx_ref, b_fx_ref, W_fh0_ref, W_fh1_ref, W_fh2_ref, W_fh3_ref,
               h_acc_ref, c_acc_ref,
               h_out_ref, c_out_ref):
    L = x_ref.shape[1]
    H = W_fx_ref.shape[1]

    x = x_ref[0]
    h0 = h0_ref[0, 0]
    c0 = c0_ref[0, 0]
    idx_d = idx_d_ref[0]
    g = g_ref[0]

    iou = (_mm(x, W_ioux_ref[...]) + b_ioux_ref[...]
           + _mm(g[3 * L:4 * L], W_iouh0_ref[...])
           + _mm(g[4 * L:], W_iouh1_ref[...]))
    i = jax.nn.sigmoid(iou[:, :H])
    o = jax.nn.sigmoid(iou[:, H:2 * H])
    u = jnp.tanh(iou[:, 2 * H:])

    f = jax.nn.sigmoid(_mm(g[:L], W_fx_ref[...]) + b_fx_ref[...]
                       + _mm(g[L:2 * L], W_fh0_ref[...] + W_fh1_ref[...])
                       + _mm(g[2 * L:3 * L], W_fh2_ref[...] + W_fh3_ref[...]))

    row_iota = lax.broadcasted_iota(jnp.int32, (L, L), 0)
    T_d = (row_iota == idx_d).astype(_F32)

    c = i * u + _mm(T_d, f * c0)

    counts = jnp.sum(T_d, axis=1, keepdims=True)
    kpos = lax.broadcasted_iota(jnp.int32, (L, 1), 0)
    upd = (counts > 0.0) & (kpos != 0)

    h = o * jnp.tanh(c)
    h_out_ref[0] = jnp.where(upd, h, h0)
    c_out_ref[0] = jnp.where(upd, c, c0)


def kernel(x, hx, tree_ids_d, tree_ids_dr, tree_ids_dl, W_ioux, b_ioux,
           W_iouh0, W_iouh1, W_fx, b_fx, W_fh0, W_fh1, W_fh2, W_fh3):
    B, L, E = x.shape
    H = W_fx.shape[1]
    n_ch = L // _CHUNK
    Bc = B // _NCHUNK

    idx_d3 = tree_ids_d.astype(jnp.int32).reshape(B, n_ch, _CHUNK)
    idx_r3 = tree_ids_dr.astype(jnp.int32).reshape(B, n_ch, _CHUNK)
    idx_l3 = tree_ids_dl.astype(jnp.int32).reshape(B, n_ch, _CHUNK)
    idx_d2 = tree_ids_d.astype(jnp.int32).reshape(B, 1, L)
    b_ioux2 = b_ioux.reshape(1, 3 * H)
    b_fx2 = b_fx.reshape(1, H)

    x2 = x.reshape(B * L, E)
    hx2 = hx.reshape(2 * B * L, H)

    def off_spec(shape, off):
        nd = len(shape)
        return pl.BlockSpec((1,) + shape[1:],
                            lambda b: (b + off,) + (0,) * (nd - 1))

    def row_spec(shape):
        nd = len(shape)
        return pl.BlockSpec((1,) + shape[1:], lambda b: (b,) + (0,) * (nd - 1))

    def full_spec(shape):
        nd = len(shape)
        return pl.BlockSpec(shape, lambda b: (0,) * nd)

    def tc_call(off):
        return pl.pallas_call(
            _cell_body,
            grid=(Bc,),
            in_specs=[
                off_spec((B, 1, L), off),
                off_spec((B, L, E), off),
                pl.BlockSpec((1, 1, L, H), lambda b: (0, b + off, 0, 0)),
                pl.BlockSpec((1, 1, L, H), lambda b: (1, b + off, 0, 0)),
                row_spec((Bc, 5 * L, H)),
                full_spec((E, 3 * H)), full_spec((1, 3 * H)),
                full_spec((H, 3 * H)), full_spec((H, 3 * H)),
                full_spec((E, H)), full_spec((1, H)),
                full_spec((H, H)), full_spec((H, H)),
                full_spec((H, H)), full_spec((H, H)),
                pl.BlockSpec((1, 8, H), lambda b: (b + off, 0, 0)),
                pl.BlockSpec((1, 8, H), lambda b: (b + off, 0, 0)),
            ],
            out_specs=[off_spec((B, L, H), off), off_spec((B, L, H), off)],
            out_shape=[jax.ShapeDtypeStruct((B, L, H), _F32),
                       jax.ShapeDtypeStruct((B, L, H), _F32)],
            input_output_aliases={15: 0, 16: 1},
        )

    h_acc = jnp.zeros((B, L, H), _F32)
    c_acc = jnp.zeros((B, L, H), _F32)
    for kc in range(_NCHUNK):
        sc_stage = _sc_stage_factory(L, H, kc * Bc, Bc)
        packed = sc_stage(x2, hx2, idx_d3, idx_r3, idx_l3)
        h_acc, c_acc = tc_call(kc * Bc)(
            idx_d2, x, hx, hx, packed.reshape(Bc, 5 * L, H),
            W_ioux, b_ioux2, W_iouh0, W_iouh1,
            W_fx, b_fx2, W_fh0, W_fh1, W_fh2, W_fh3,
            h_acc, c_acc)
    return (h_acc, c_acc)


# speedup vs baseline: 1.4233x; 1.4233x over previous
import functools

import jax
import jax.numpy as jnp
from jax import lax
from jax.experimental import pallas as pl
from jax.experimental.pallas import tpu as pltpu
from jax.experimental.pallas import tpu_sc as plsc

_F32 = jnp.float32

_NC = 2
_NS = 16
_LN = 16
_NW = _NC * _NS
_CHUNK = 128
_NCHUNK = 8


def _mm(a, b):
    return lax.dot_general(a, b, (((1,), (0,)), ((), ())),
                           preferred_element_type=_F32)


def _sc_stage_factory(L, H, off, Bc):
    n_ch = L // _CHUNK
    rows_per_w = Bc // _NW
    mesh = plsc.VectorSubcoreMesh(core_axis_name="c", subcore_axis_name="s")

    @functools.partial(
        pl.kernel,
        out_type=jax.ShapeDtypeStruct((Bc * 5 * L, H), _F32),
        mesh=mesh,
        scratch_types=[
            pltpu.VMEM((n_ch, _CHUNK), jnp.int32),
            pltpu.VMEM((n_ch, _CHUNK), jnp.int32),
            pltpu.VMEM((n_ch, _CHUNK), jnp.int32),
            pltpu.VMEM((n_ch, _CHUNK), jnp.int32),
            pltpu.VMEM((n_ch, _CHUNK), jnp.int32),
            pltpu.VMEM((_CHUNK, H), _F32),
            pltpu.VMEM((_CHUNK, H), _F32),
            pltpu.VMEM((_CHUNK, H), _F32),
            pltpu.VMEM_SHARED((_NS * L, H), _F32),
            pltpu.SemaphoreType.DMA,
            pltpu.SemaphoreType.DMA,
            pltpu.SemaphoreType.DMA,
            pltpu.SemaphoreType.DMA,
            pltpu.SemaphoreType.DMA,
            pltpu.SemaphoreType.DMA,
            pltpu.SemaphoreType.DMA,
            pltpu.SemaphoreType.DMA,
        ],
    )
    def sc_stage(x_hbm, h0_hbm, idxd_hbm, idxr_hbm, idxl_hbm, out_hbm,
                 gidx_d, gidx_r, gidx_l, sidx_r, sidx_l,
                 G0, G1, zbuf, acc,
                 isem, gsemA, gsemB, wsemA, wsemB, zsem, ssemA, ssemB):
        cid = lax.axis_index("c")
        sid = lax.axis_index("s")
        w = sid * _NC + cid
        accbase = sid * L
        G = (G0, G1)
        gsem = (gsemA, gsemB)
        wsem = (wsemA, wsemB)
        ssem = (ssemA, ssemB)

        def zfill(j, carry):
            for k in range(H // _LN):
                zbuf[j, pl.ds(k * _LN, _LN)] = jnp.zeros((_LN,), _F32)
            return carry
        lax.fori_loop(0, _CHUNK, zfill, 0)

        def row_body(t, carry):
            b = off + w * rows_per_w + t
            bL = b * L
            ob = (w * rows_per_w + t) * 5 * L

            i0 = pltpu.async_copy(idxd_hbm.at[b], gidx_d, isem)
            i1 = pltpu.async_copy(idxr_hbm.at[b], gidx_r, isem)
            i2 = pltpu.async_copy(idxl_hbm.at[b], gidx_l, isem)
            i0.wait(); i1.wait(); i2.wait()
            for j in range(n_ch):
                for k in range(_CHUNK // _LN):
                    sl = pl.ds(k * _LN, _LN)
                    sidx_r[j, sl] = gidx_r[j, sl] + accbase
                    sidx_l[j, sl] = gidx_l[j, sl] + accbase
                    gidx_d[j, sl] = gidx_d[j, sl] + bL
                    gidx_r[j, sl] = gidx_r[j, sl] + bL
                    gidx_l[j, sl] = gidx_l[j, sl] + bL

            zs = [pltpu.async_copy(
                      zbuf, acc.at[pl.ds(accbase + ch * _CHUNK, _CHUNK)], zsem)
                  for ch in range(n_ch)]
            for z in zs:
                z.wait()

            def scatter(si):
                sds = {}
                ads = {}
                for ch in range(n_ch):
                    bid = ch % 2
                    if ch >= 2:
                        ads[ch - 2].wait()
                    sds[ch] = pltpu.async_copy(
                        h0_hbm.at[pl.ds(bL + ch * _CHUNK, _CHUNK)],
                        G[bid], gsem[bid])
                    sds[ch].wait()
                    ads[ch] = pltpu.async_copy(
                        G[bid], acc.at[si.at[ch]], ssem[bid], add=True)
                ads[n_ch - 2].wait()
                ads[n_ch - 1].wait()

            scatter(sidx_r)
            ro_r = pltpu.async_copy(acc.at[pl.ds(accbase, L)],
                                    out_hbm.at[pl.ds(ob + 3 * L, L)], wsemA)
            ro_r.wait()
            zs = [pltpu.async_copy(
                      zbuf, acc.at[pl.ds(accbase + ch * _CHUNK, _CHUNK)], zsem)
                  for ch in range(n_ch)]
            for z in zs:
                z.wait()
            scatter(sidx_l)
            ro_l = pltpu.async_copy(acc.at[pl.ds(accbase, L)],
                                    out_hbm.at[pl.ds(ob + 4 * L, L)], wsemB)

            units = []
            for k, (gi, src) in enumerate(((gidx_d, x_hbm), (gidx_r, h0_hbm),
                                           (gidx_l, h0_hbm))):
                for ch in range(n_ch):
                    units.append((gi, src, ob + k * L + ch * _CHUNK))
            gds = {}
            wds = {}

            def writeout(u):
                return pltpu.async_copy(
                    G[u % 2], out_hbm.at[pl.ds(units[u][2], _CHUNK)],
                    wsem[u % 2])

            for u, (gi, src, _) in enumerate(units):
                bid = u % 2
                if u >= 2:
                    wds[u - 2].wait()
                gds[u] = pltpu.async_copy(
                    src.at[gi.at[u % n_ch]], G[bid], gsem[bid])
                if u >= 1:
                    gds[u - 1].wait()
                    wds[u - 1] = writeout(u - 1)
            last = len(units) - 1
            gds[last].wait()
            wds[last] = writeout(last)
            wds[last - 1].wait()
            wds[last].wait()
            ro_l.wait()
            return carry

        lax.fori_loop(0, rows_per_w, row_body, 0)

    return sc_stage


def _cell_body(idx_d_ref, x_ref, h0_ref, c0_ref, g_ref,
               W_ioux_ref, b_ioux_ref, W_iouh0_ref, W_iouh1_ref,
               W_fx_ref, b_fx_ref, W_fh0_ref, W_fh1_ref, W_fh2_ref, W_fh3_ref,
               h_acc_ref, c_acc_ref,
               h_out_ref, c_out_ref):
    L = x_ref.shape[1]
    H = W_fx_ref.shape[1]

    x = x_ref[0]
    h0 = h0_ref[0, 0]
    c0 = c0_ref[0, 0]
    idx_d = idx_d_ref[0]
    g = g_ref[0]

    iou = (_mm(x, W_ioux_ref[...]) + b_ioux_ref[...]
           + _mm(g[3 * L:4 * L], W_iouh0_ref[...])
           + _mm(g[4 * L:], W_iouh1_ref[...]))
    i = jax.nn.sigmoid(iou[:, :H])
    o = jax.nn.sigmoid(iou[:, H:2 * H])
    u = jnp.tanh(iou[:, 2 * H:])

    f = jax.nn.sigmoid(_mm(g[:L], W_fx_ref[...]) + b_fx_ref[...]
                       + _mm(g[L:2 * L], W_fh0_ref[...] + W_fh1_ref[...])
                       + _mm(g[2 * L:3 * L], W_fh2_ref[...] + W_fh3_ref[...]))

    row_iota = lax.broadcasted_iota(jnp.int32, (L, L), 0)
    T_d = (row_iota == idx_d).astype(_F32)

    c = i * u + _mm(T_d, f * c0)

    counts = jnp.sum(T_d, axis=1, keepdims=True)
    kpos = lax.broadcasted_iota(jnp.int32, (L, 1), 0)
    upd = (counts > 0.0) & (kpos != 0)

    h = o * jnp.tanh(c)
    h_out_ref[0] = jnp.where(upd, h, h0)
    c_out_ref[0] = jnp.where(upd, c, c0)


def kernel(x, hx, tree_ids_d, tree_ids_dr, tree_ids_dl, W_ioux, b_ioux,
           W_iouh0, W_iouh1, W_fx, b_fx, W_fh0, W_fh1, W_fh2, W_fh3):
    B, L, E = x.shape
    H = W_fx.shape[1]
    n_ch = L // _CHUNK
    Bc = B // _NCHUNK

    idx_d3 = tree_ids_d.astype(jnp.int32).reshape(B, n_ch, _CHUNK)
    idx_r3 = tree_ids_dr.astype(jnp.int32).reshape(B, n_ch, _CHUNK)
    idx_l3 = tree_ids_dl.astype(jnp.int32).reshape(B, n_ch, _CHUNK)
    idx_d2 = tree_ids_d.astype(jnp.int32).reshape(B, 1, L)
    b_ioux2 = b_ioux.reshape(1, 3 * H)
    b_fx2 = b_fx.reshape(1, H)

    x2 = x.reshape(B * L, E)
    hx2 = hx.reshape(2 * B * L, H)

    def off_spec(shape, off):
        nd = len(shape)
        return pl.BlockSpec((1,) + shape[1:],
                            lambda b: (b + off,) + (0,) * (nd - 1))

    def row_spec(shape):
        nd = len(shape)
        return pl.BlockSpec((1,) + shape[1:], lambda b: (b,) + (0,) * (nd - 1))

    def full_spec(shape):
        nd = len(shape)
        return pl.BlockSpec(shape, lambda b: (0,) * nd)

    def tc_call(off):
        return pl.pallas_call(
            _cell_body,
            grid=(Bc,),
            in_specs=[
                off_spec((B, 1, L), off),
                off_spec((B, L, E), off),
                pl.BlockSpec((1, 1, L, H), lambda b: (0, b + off, 0, 0)),
                pl.BlockSpec((1, 1, L, H), lambda b: (1, b + off, 0, 0)),
                row_spec((Bc, 5 * L, H)),
                full_spec((E, 3 * H)), full_spec((1, 3 * H)),
                full_spec((H, 3 * H)), full_spec((H, 3 * H)),
                full_spec((E, H)), full_spec((1, H)),
                full_spec((H, H)), full_spec((H, H)),
                full_spec((H, H)), full_spec((H, H)),
                pl.BlockSpec((1, 8, H), lambda b: (b + off, 0, 0)),
                pl.BlockSpec((1, 8, H), lambda b: (b + off, 0, 0)),
            ],
            out_specs=[off_spec((B, L, H), off), off_spec((B, L, H), off)],
            out_shape=[jax.ShapeDtypeStruct((B, L, H), _F32),
                       jax.ShapeDtypeStruct((B, L, H), _F32)],
            input_output_aliases={15: 0, 16: 1},
        )

    h_acc = jnp.zeros((B, L, H), _F32)
    c_acc = jnp.zeros((B, L, H), _F32)
    for kc in range(_NCHUNK):
        sc_stage = _sc_stage_factory(L, H, kc * Bc, Bc)
        packed = sc_stage(x2, hx2, idx_d3, idx_r3, idx_l3)
        h_acc, c_acc = tc_call(kc * Bc)(
            idx_d2, x, hx, hx, packed.reshape(Bc, 5 * L, H),
            W_ioux, b_ioux2, W_iouh0, W_iouh1,
            W_fx, b_fx2, W_fh0, W_fh1, W_fh2, W_fh3,
            h_acc, c_acc)
    return (h_acc, c_acc)
